# baseline (speedup 1.0000x reference)
import functools

import jax
import jax.numpy as jnp
from jax import lax
from jax.experimental import pallas as pl
from jax.experimental.pallas import tpu as pltpu
from jax.experimental.pallas import tpu_sc as plsc

_NUM_TREES = 2048
_NODES_PER_TREE = 512
_DEPTH = 8
_NFEAT = 512
_NCLS = 10
_L = 16
_NW = 32
_TPW = _NUM_TREES // _NW
_G = _TPW // _L

_mesh = plsc.VectorSubcoreMesh(core_axis_name="c", subcore_axis_name="s")


@functools.partial(
    pl.kernel,
    out_type=jax.ShapeDtypeStruct((_NW, _L), jnp.float32),
    mesh=_mesh,
    compiler_params=pltpu.CompilerParams(
        needs_layout_passes=False, use_tc_tiling_on_sc=False),
    scratch_types=[
        pltpu.VMEM((_NFEAT,), jnp.float32),
        pltpu.VMEM((_TPW,), jnp.int32),
        pltpu.VMEM((_TPW,), jnp.int32),
        pltpu.VMEM((_TPW,), jnp.int32),
        pltpu.VMEM((_TPW,), jnp.float32),
        pltpu.VMEM((_TPW,), jnp.float32),
        pltpu.VMEM((_TPW,), jnp.float32),
        pltpu.VMEM((_NCLS, _TPW), jnp.float32),
        pltpu.VMEM((_L,), jnp.float32),
        pltpu.SemaphoreType.DMA,
    ],
)
def _traverse_sc(x_hbm, feat_hbm, thr_hbm, left_hbm, right_hbm, vt_hbm,
                 roots_hbm, out_hbm, x_v, off_v, idx_v, feat_v, thr_v,
                 left_v, right_v, vals_v, acc_v, sem):
    cid = lax.axis_index("c")
    sid = lax.axis_index("s")
    wid = sid * 2 + cid
    base = wid * _TPW

    pltpu.sync_copy(x_hbm, x_v)
    pltpu.sync_copy(roots_hbm.at[pl.ds(base, _TPW)], off_v)
    pltpu.sync_copy(roots_hbm.at[pl.ds(base, _TPW)], idx_v)

    for _ in range(_DEPTH):
        c1 = pltpu.async_copy(feat_hbm.at[idx_v], feat_v, sem)
        c2 = pltpu.async_copy(thr_hbm.at[idx_v], thr_v, sem)
        c3 = pltpu.async_copy(left_hbm.at[idx_v], left_v, sem)
        c4 = pltpu.async_copy(right_hbm.at[idx_v], right_v, sem)
        c1.wait()
        c2.wait()
        c3.wait()
        c4.wait()
        for g in range(_G):
            sl = pl.ds(g * _L, _L)
            f = feat_v[sl]
            xv = plsc.load_gather(x_v, [f])
            go_left = xv < thr_v[sl]
            nxt = jnp.where(go_left, left_v[sl], right_v[sl]).astype(jnp.int32)
            idx_v[sl] = nxt + off_v[sl]

    vcps = [pltpu.async_copy(vt_hbm.at[c].at[idx_v], vals_v.at[c], sem)
            for c in range(_NCLS)]
    for cp in vcps:
        cp.wait()

    lane = lax.iota(jnp.int32, _L)
    acc = jnp.zeros((_L,), jnp.float32)
    for c in range(_NCLS):
        s = jnp.zeros((_L,), jnp.float32)
        for g in range(_G):
            s = s + vals_v[c, pl.ds(g * _L, _L)]
        total = lax.reduce_sum_p.bind(s, axes=(0,))
        acc = acc + jnp.where(lane == c, total, 0.0)
    acc_v[...] = acc
    pltpu.sync_copy(acc_v, out_hbm.at[wid])


def kernel(x, features, thresholds, lefts, rights, values, nodes_offset, indices):
    partials = _traverse_sc(x.reshape(-1), features, thresholds, lefts,
                            rights, values.T, indices)
    return jnp.sum(partials, axis=0)[:_NCLS].reshape(1, _NCLS)


# speedup vs baseline: 19.0781x; 19.0781x over previous
import functools

import jax
import jax.numpy as jnp
from jax import lax
from jax.experimental import pallas as pl
from jax.experimental.pallas import tpu as pltpu
from jax.experimental.pallas import tpu_sc as plsc

_NUM_TREES = 2048
_NODES_PER_TREE = 512
_DEPTH = 8
_NFEAT = 512
_NCLS = 10
_L = 16
_NW = 32
_TPW = _NUM_TREES // _NW
_G = _TPW // _L

_mesh = plsc.VectorSubcoreMesh(core_axis_name="c", subcore_axis_name="s")


@functools.partial(
    pl.kernel,
    out_type=jax.ShapeDtypeStruct((_NW, _L), jnp.float32),
    mesh=_mesh,
    compiler_params=pltpu.CompilerParams(needs_layout_passes=False),
    scratch_types=[
        pltpu.VMEM((_NFEAT,), jnp.float32),
        pltpu.VMEM((_TPW,), jnp.int32),
        pltpu.VMEM((_TPW,), jnp.int32),
        pltpu.VMEM((_TPW,), jnp.int32),
        pltpu.VMEM((_TPW,), jnp.float32),
        pltpu.VMEM((_TPW,), jnp.float32),
        pltpu.VMEM((_TPW,), jnp.float32),
        pltpu.VMEM((2, _L, _NCLS, 128), jnp.float32),
        pltpu.VMEM((_L,), jnp.float32),
        pltpu.SemaphoreType.DMA,
        pltpu.SemaphoreType.DMA,
        pltpu.SemaphoreType.DMA,
    ],
)
def _traverse_sc(x_hbm, feat_hbm, thr_hbm, left_hbm, right_hbm, vt_hbm,
                 roots_hbm, out_hbm, x_v, off_v, idx_v, feat_v, thr_v,
                 left_v, right_v, vbuf, acc_v, sem, semv0, semv1):
    cid = lax.axis_index("c")
    sid = lax.axis_index("s")
    wid = sid * 2 + cid
    base = wid * _TPW

    pltpu.sync_copy(x_hbm, x_v)
    pltpu.sync_copy(roots_hbm.at[pl.ds(base, _TPW)], off_v)
    pltpu.sync_copy(roots_hbm.at[pl.ds(base, _TPW)], idx_v)

    for _ in range(_DEPTH):
        c1 = pltpu.async_copy(feat_hbm.at[idx_v], feat_v, sem)
        c2 = pltpu.async_copy(thr_hbm.at[idx_v], thr_v, sem)
        c3 = pltpu.async_copy(left_hbm.at[idx_v], left_v, sem)
        c4 = pltpu.async_copy(right_hbm.at[idx_v], right_v, sem)
        c1.wait()
        c2.wait()
        c3.wait()
        c4.wait()
        for g in range(_G):
            sl = pl.ds(g * _L, _L)
            f = feat_v[sl]
            xv = plsc.load_gather(x_v, [f])
            go_left = xv < thr_v[sl]
            nxt = jnp.where(go_left, left_v[sl], right_v[sl]).astype(jnp.int32)
            idx_v[sl] = nxt + off_v[sl]

    lane = lax.iota(jnp.int32, _L)
    semv = (semv0, semv1)

    def fire_wave(w):
        grp = idx_v[pl.ds(w * _L, _L)]
        cps, cols = [], []
        for j in range(_L):
            sc_idx = lax.reduce_sum_p.bind(
                jnp.where(lane == j, grp, 0), axes=(0,))
            aligned = pl.multiple_of(
                lax.shift_left(lax.shift_right_logical(sc_idx, 7), 7), 128)
            cps.append(pltpu.async_copy(
                vt_hbm.at[:, pl.ds(aligned, 128)],
                vbuf.at[w % 2].at[j], semv[w % 2]))
            cols.append(jnp.bitwise_and(sc_idx, 127))
        return cps, cols

    cidx = jnp.minimum(lane, _NCLS - 1)
    acc = jnp.zeros((_L,), jnp.float32)
    waves = [fire_wave(0), fire_wave(1)]
    for w in range(_G):
        cps, cols = waves[w % 2]
        for cp in cps:
            cp.wait()
        for j in range(_L):
            col = jnp.full((_L,), cols[j], jnp.int32)
            acc = acc + plsc.load_gather(vbuf.at[w % 2].at[j], [cidx, col])
        if w + 2 < _G:
            waves[w % 2] = fire_wave(w + 2)
    acc = jnp.where(lane < _NCLS, acc, 0.0)
    acc_v[...] = acc
    pltpu.sync_copy(acc_v, out_hbm.at[wid])


def kernel(x, features, thresholds, lefts, rights, values, nodes_offset, indices):
    partials = _traverse_sc(x.reshape(-1), features, thresholds, lefts,
                            rights, values.T, indices)
    return jnp.sum(partials, axis=0)[:_NCLS].reshape(1, _NCLS)


# speedup vs baseline: 19.8969x; 1.0429x over previous
import functools

import jax
import jax.numpy as jnp
from jax import lax
from jax.experimental import pallas as pl
from jax.experimental.pallas import tpu as pltpu
from jax.experimental.pallas import tpu_sc as plsc

_NUM_TREES = 2048
_NODES_PER_TREE = 512
_DEPTH = 8
_NFEAT = 512
_NCLS = 10
_L = 16
_NW = 32
_TPW = _NUM_TREES // _NW
_G = _TPW // _L
_H = _TPW // 2
_GH = _H // _L

_mesh = plsc.VectorSubcoreMesh(core_axis_name="c", subcore_axis_name="s")


@functools.partial(
    pl.kernel,
    out_type=jax.ShapeDtypeStruct((_NW, _L), jnp.float32),
    mesh=_mesh,
    compiler_params=pltpu.CompilerParams(needs_layout_passes=False),
    scratch_types=[
        pltpu.VMEM((_NFEAT,), jnp.float32),
        pltpu.VMEM((_TPW,), jnp.int32),
        pltpu.VMEM((_TPW,), jnp.int32),
        pltpu.VMEM((2, _H), jnp.int32),
        pltpu.VMEM((2, _H), jnp.float32),
        pltpu.VMEM((2, _H), jnp.float32),
        pltpu.VMEM((2, _H), jnp.float32),
        pltpu.VMEM((2, _L, _NCLS, 128), jnp.float32),
        pltpu.VMEM((_L,), jnp.float32),
        pltpu.SemaphoreType.DMA,
        pltpu.SemaphoreType.DMA,
        pltpu.SemaphoreType.DMA,
        pltpu.SemaphoreType.DMA,
    ],
)
def _traverse_sc(x_hbm, feat_hbm, thr_hbm, left_hbm, right_hbm, vt_hbm,
                 roots_hbm, out_hbm, x_v, off_v, idx_v, feat_b, thr_b,
                 left_b, right_b, vbuf, acc_v, sem0, sem1, semv0, semv1):
    cid = lax.axis_index("c")
    sid = lax.axis_index("s")
    wid = sid * 2 + cid
    base = wid * _TPW
    sems = (sem0, sem1)

    pltpu.sync_copy(roots_hbm.at[pl.ds(base, _TPW)], idx_v)

    def fire(h):
        isl = idx_v.at[pl.ds(h * _H, _H)]
        return [
            pltpu.async_copy(feat_hbm.at[isl], feat_b.at[h], sems[h]),
            pltpu.async_copy(thr_hbm.at[isl], thr_b.at[h], sems[h]),
            pltpu.async_copy(left_hbm.at[isl], left_b.at[h], sems[h]),
            pltpu.async_copy(right_hbm.at[isl], right_b.at[h], sems[h]),
        ]

    cps = [fire(0), fire(1)]
    pltpu.sync_copy(x_hbm, x_v)
    pltpu.sync_copy(roots_hbm.at[pl.ds(base, _TPW)], off_v)

    def compute(h):
        for g in range(_GH):
            sl = pl.ds(g * _L, _L)
            dsl = pl.ds(h * _H + g * _L, _L)
            f = feat_b[h, sl]
            xv = plsc.load_gather(x_v, [f])
            go_left = xv < thr_b[h, sl]
            nxt = jnp.where(go_left, left_b[h, sl],
                            right_b[h, sl]).astype(jnp.int32)
            idx_v[dsl] = nxt + off_v[dsl]

    for l in range(_DEPTH):
        for h in (0, 1):
            for cp in cps[h]:
                cp.wait()
            compute(h)
            if l < _DEPTH - 1:
                cps[h] = fire(h)

    lane = lax.iota(jnp.int32, _L)
    semv = (semv0, semv1)

    def fire_wave(w):
        grp = idx_v[pl.ds(w * _L, _L)]
        wcps, cols = [], []
        for j in range(_L):
            sc_idx = lax.reduce_sum_p.bind(
                jnp.where(lane == j, grp, 0), axes=(0,))
            aligned = pl.multiple_of(
                lax.shift_left(lax.shift_right_logical(sc_idx, 7), 7), 128)
            wcps.append(pltpu.async_copy(
                vt_hbm.at[:, pl.ds(aligned, 128)],
                vbuf.at[w % 2].at[j], semv[w % 2]))
            cols.append(jnp.bitwise_and(sc_idx, 127))
        return wcps, cols

    cidx = jnp.minimum(lane, _NCLS - 1)
    acc = jnp.zeros((_L,), jnp.float32)
    waves = [fire_wave(0), fire_wave(1)]
    for w in range(_G):
        wcps, cols = waves[w % 2]
        for cp in wcps:
            cp.wait()
        for j in range(_L):
            col = jnp.full((_L,), cols[j], jnp.int32)
            acc = acc + plsc.load_gather(vbuf.at[w % 2].at[j], [cidx, col])
        if w + 2 < _G:
            waves[w % 2] = fire_wave(w + 2)
    acc = jnp.where(lane < _NCLS, acc, 0.0)
    acc_v[...] = acc
    pltpu.sync_copy(acc_v, out_hbm.at[wid])


def kernel(x, features, thresholds, lefts, rights, values, nodes_offset, indices):
    partials = _traverse_sc(x.reshape(-1), features, thresholds, lefts,
                            rights, values.T, indices)
    return jnp.sum(partials, axis=0)[:_NCLS].reshape(1, _NCLS)


# speedup vs baseline: 20.0173x; 1.0060x over previous
import functools

import jax
import jax.numpy as jnp
from jax import lax
from jax.experimental import pallas as pl
from jax.experimental.pallas import tpu as pltpu
from jax.experimental.pallas import tpu_sc as plsc

_NUM_TREES = 2048
_NODES_PER_TREE = 512
_DEPTH = 8
_NFEAT = 512
_NCLS = 10
_L = 16
_NW = 32
_TPW = _NUM_TREES // _NW
_G = _TPW // _L
_H = _TPW // 2
_GH = _H // _L

_mesh = plsc.VectorSubcoreMesh(core_axis_name="c", subcore_axis_name="s")


@functools.partial(
    pl.kernel,
    out_type=jax.ShapeDtypeStruct((_NW, _L), jnp.float32),
    mesh=_mesh,
    compiler_params=pltpu.CompilerParams(needs_layout_passes=False),
    scratch_types=[
        pltpu.VMEM((_NFEAT,), jnp.float32),
        pltpu.VMEM((_TPW,), jnp.int32),
        pltpu.VMEM((_TPW,), jnp.int32),
        pltpu.VMEM((2, _H), jnp.int32),
        pltpu.VMEM((2, _H), jnp.float32),
        pltpu.VMEM((2, _H), jnp.float32),
        pltpu.VMEM((2, _H), jnp.float32),
        pltpu.VMEM((2, _L, _NCLS, 128), jnp.float32),
        pltpu.VMEM((_L,), jnp.float32),
        pltpu.SemaphoreType.DMA,
        pltpu.SemaphoreType.DMA,
        pltpu.SemaphoreType.DMA,
        pltpu.SemaphoreType.DMA,
    ],
)
def _traverse_sc(x_hbm, feat_hbm, thr_hbm, left_hbm, right_hbm, vt_hbm,
                 roots_hbm, out_hbm, x_v, off_v, idx_v, feat_b, thr_b,
                 left_b, right_b, vbuf, acc_v, sem0, sem1, semv0, semv1):
    cid = lax.axis_index("c")
    sid = lax.axis_index("s")
    wid = sid * 2 + cid
    base = wid * _TPW
    sems = (sem0, sem1)

    pltpu.sync_copy(roots_hbm.at[pl.ds(base, _TPW)], idx_v)

    def fire(h):
        isl = idx_v.at[pl.ds(h * _H, _H)]
        return [
            pltpu.async_copy(feat_hbm.at[isl], feat_b.at[h], sems[h]),
            pltpu.async_copy(thr_hbm.at[isl], thr_b.at[h], sems[h]),
            pltpu.async_copy(left_hbm.at[isl], left_b.at[h], sems[h]),
            pltpu.async_copy(right_hbm.at[isl], right_b.at[h], sems[h]),
        ]

    cps = [fire(0), fire(1)]
    pltpu.sync_copy(x_hbm, x_v)
    pltpu.sync_copy(roots_hbm.at[pl.ds(base, _TPW)], off_v)

    def compute(h):
        for g in range(_GH):
            sl = pl.ds(g * _L, _L)
            dsl = pl.ds(h * _H + g * _L, _L)
            f = feat_b[h, sl]
            xv = plsc.load_gather(x_v, [f])
            go_left = xv < thr_b[h, sl]
            nxt = jnp.where(go_left, left_b[h, sl],
                            right_b[h, sl]).astype(jnp.int32)
            idx_v[dsl] = nxt + off_v[dsl]

    lane = lax.iota(jnp.int32, _L)
    semv = (semv0, semv1)

    def fire_wave(w):
        grp = idx_v[pl.ds(w * _L, _L)]
        wcps, cols = [], []
        for j in range(_L):
            sc_idx = lax.reduce_sum_p.bind(
                jnp.where(lane == j, grp, 0), axes=(0,))
            aligned = pl.multiple_of(
                lax.shift_left(lax.shift_right_logical(sc_idx, 7), 7), 128)
            wcps.append(pltpu.async_copy(
                vt_hbm.at[:, pl.ds(aligned, 128)],
                vbuf.at[w % 2].at[j], semv[w % 2]))
            cols.append(jnp.bitwise_and(sc_idx, 127))
        return wcps, cols

    waves = [None, None]
    for l in range(_DEPTH):
        for h in (0, 1):
            for cp in cps[h]:
                cp.wait()
            compute(h)
            if l < _DEPTH - 1:
                cps[h] = fire(h)
            elif h == 0:
                waves[0] = fire_wave(0)
                waves[1] = fire_wave(1)

    cidx = jnp.minimum(lane, _NCLS - 1)
    acc = jnp.zeros((_L,), jnp.float32)
    for w in range(_G):
        wcps, cols = waves[w % 2]
        for cp in wcps:
            cp.wait()
        for j in range(_L):
            col = jnp.full((_L,), cols[j], jnp.int32)
            acc = acc + plsc.load_gather(vbuf.at[w % 2].at[j], [cidx, col])
        if w + 2 < _G:
            waves[w % 2] = fire_wave(w + 2)
    acc = jnp.where(lane < _NCLS, acc, 0.0)
    acc_v[...] = acc
    pltpu.sync_copy(acc_v, out_hbm.at[wid])


def kernel(x, features, thresholds, lefts, rights, values, nodes_offset, indices):
    partials = _traverse_sc(x.reshape(-1), features, thresholds, lefts,
                            rights, values.T, indices)
    return jnp.sum(partials, axis=0)[:_NCLS].reshape(1, _NCLS)
